# baseline (speedup 1.0000x reference)
import jax
import jax.numpy as jnp
from jax.experimental import pallas as pl
from jax.experimental.pallas import tpu as pltpu

_NUM_EMB = 1024
_DIM = 64
_N = 8 * 576
_K = 4
_R = _N // _K


def _vq_kernel(x_ref, w_ref, q_hbm, enc_hbm, idx_hbm, loss_ref,
               enc_buf, q_buf, idx_buf, w2_acc, counts_acc, mse_acc, sem):
    i = pl.program_id(0)
    slot = jax.lax.rem(i, 2)

    def enc_copy(step, s):
        return pltpu.make_async_copy(
            enc_buf.at[s], enc_hbm.at[pl.ds(step * _R, _R), :], sem.at[0, s])

    def q_copy(step, s):
        return pltpu.make_async_copy(
            q_buf.at[s], q_hbm.at[pl.ds(step * _R, _R), :], sem.at[1, s])

    def idx_copy(step, s):
        return pltpu.make_async_copy(
            idx_buf.at[s], idx_hbm.at[step], sem.at[2, s])

    @pl.when(i >= 2)
    def _wait_prev():
        enc_copy(i - 2, slot).wait()
        q_copy(i - 2, slot).wait()
        idx_copy(i - 2, slot).wait()

    xb = x_ref[...]
    w = w_ref[...]

    @pl.when(i == 0)
    def _w2():
        w2_acc[...] = jnp.sum(w * w, axis=1)[None, :]

    x2 = jnp.sum(xb * xb, axis=1, keepdims=True)
    xw = jax.lax.dot_general(xb, w, (((1,), (1,)), ((), ())),
                             preferred_element_type=jnp.float32)
    d2 = x2 + w2_acc[...] - 2.0 * xw

    idx = jnp.argmin(d2, axis=1).astype(jnp.int32)
    idx_buf[slot, 0] = idx

    cols = jax.lax.broadcasted_iota(jnp.int32, d2.shape, 1)
    one_hot = (cols == idx[:, None]).astype(jnp.float32)
    enc_buf[slot] = one_hot

    q = jax.lax.dot_general(one_hot, w, (((1,), (0,)), ((), ())),
                            preferred_element_type=jnp.float32)
    q_buf[slot] = q

    enc_copy(i, slot).start()
    q_copy(i, slot).start()
    idx_copy(i, slot).start()

    diff = q - xb
    mse_part = jnp.sum(diff * diff).reshape(1, 1)
    counts_part = jnp.sum(one_hot, axis=0, keepdims=True)

    @pl.when(i == 0)
    def _init():
        counts_acc[...] = counts_part
        mse_acc[...] = mse_part

    @pl.when(i != 0)
    def _acc():
        counts_acc[...] += counts_part
        mse_acc[...] += mse_part

    @pl.when(i == _K - 1)
    def _finalize():
        usage = counts_acc[...] * (1.0 / _N)
        du = usage - (1.0 / _NUM_EMB)
        usage_loss = jnp.sum(du * du) * (1.0 / _NUM_EMB)
        mse = mse_acc[...] * (1.0 / (_N * _DIM))
        loss_ref[...] = 1.25 * mse + 0.1 * usage_loss
        enc_copy(i - 1, 1 - slot).wait()
        q_copy(i - 1, 1 - slot).wait()
        idx_copy(i - 1, 1 - slot).wait()
        enc_copy(i, slot).wait()
        q_copy(i, slot).wait()
        idx_copy(i, slot).wait()


@jax.jit
def kernel(x, W):
    b, l, d = x.shape
    n = b * l
    flat = x.reshape(n, d)
    out_types = (
        jax.ShapeDtypeStruct((n, d), jnp.float32),
        jax.ShapeDtypeStruct((n, _NUM_EMB), jnp.float32),
        jax.ShapeDtypeStruct((_K, 1, _R), jnp.int32),
        jax.ShapeDtypeStruct((1, 1), jnp.float32),
    )
    q, enc, idx, loss = pl.pallas_call(
        _vq_kernel,
        grid=(_K,),
        in_specs=[
            pl.BlockSpec((_R, d), lambda i: (i, 0)),
            pl.BlockSpec((_NUM_EMB, d), lambda i: (0, 0)),
        ],
        out_specs=(
            pl.BlockSpec(memory_space=pltpu.HBM),
            pl.BlockSpec(memory_space=pltpu.HBM),
            pl.BlockSpec(memory_space=pltpu.HBM),
            pl.BlockSpec((1, 1), lambda i: (0, 0)),
        ),
        out_shape=out_types,
        scratch_shapes=[
            pltpu.VMEM((2, _R, _NUM_EMB), jnp.float32),
            pltpu.VMEM((2, _R, _DIM), jnp.float32),
            pltpu.VMEM((2, 1, _R), jnp.int32),
            pltpu.VMEM((1, _NUM_EMB), jnp.float32),
            pltpu.VMEM((1, _NUM_EMB), jnp.float32),
            pltpu.VMEM((1, 1), jnp.float32),
            pltpu.SemaphoreType.DMA((3, 2)),
        ],
    )(flat, W)
    return (q.reshape(b, l, d), loss.reshape(()),
            enc.reshape(b, l, _NUM_EMB), idx.reshape(b, l))


# speedup vs baseline: 1.2088x; 1.2088x over previous
import jax
import jax.numpy as jnp
from jax.experimental import pallas as pl
from jax.experimental.pallas import tpu as pltpu

_NUM_EMB = 1024
_DIM = 64
_N = 8 * 576
_K = 4
_R = _N // _K


def _vq_kernel(x_ref, w_ref, q_hbm, enc_hbm, idx_ref, loss_ref,
               enc_buf, q_buf, sem):
    w = w_ref[...]
    w2 = jnp.sum(w * w, axis=1)[None, :]

    counts = None
    mse = None
    for c in range(_K):
        xb = x_ref[c * _R:(c + 1) * _R, :]
        x2 = jnp.sum(xb * xb, axis=1, keepdims=True)
        xw = jax.lax.dot_general(xb, w, (((1,), (1,)), ((), ())),
                                 preferred_element_type=jnp.float32)
        d2 = x2 + w2 - 2.0 * xw

        idx = jnp.argmin(d2, axis=1).astype(jnp.int32)
        idx_ref[c, 0] = idx

        cols = jax.lax.broadcasted_iota(jnp.int32, d2.shape, 1)
        one_hot = (cols == idx[:, None]).astype(jnp.float32)
        enc_buf[c] = one_hot
        pltpu.make_async_copy(
            enc_buf.at[c], enc_hbm.at[pl.ds(c * _R, _R), :],
            sem.at[0, c]).start()

        q = jax.lax.dot_general(one_hot, w, (((1,), (0,)), ((), ())),
                                preferred_element_type=jnp.float32)
        q_buf[c] = q
        pltpu.make_async_copy(
            q_buf.at[c], q_hbm.at[pl.ds(c * _R, _R), :],
            sem.at[1, c]).start()

        diff = q - xb
        mse_part = jnp.sum(diff * diff).reshape(1, 1)
        counts_part = jnp.sum(one_hot, axis=0, keepdims=True)
        counts = counts_part if c == 0 else counts + counts_part
        mse = mse_part if c == 0 else mse + mse_part

    usage = counts * (1.0 / _N)
    du = usage - (1.0 / _NUM_EMB)
    usage_loss = jnp.sum(du * du) * (1.0 / _NUM_EMB)
    loss_ref[...] = 1.25 * (mse * (1.0 / (_N * _DIM))) + 0.1 * usage_loss

    for c in range(_K):
        pltpu.make_async_copy(
            enc_buf.at[c], enc_hbm.at[pl.ds(c * _R, _R), :],
            sem.at[0, c]).wait()
        pltpu.make_async_copy(
            q_buf.at[c], q_hbm.at[pl.ds(c * _R, _R), :],
            sem.at[1, c]).wait()


@jax.jit
def kernel(x, W):
    b, l, d = x.shape
    n = b * l
    flat = x.reshape(n, d)
    out_types = (
        jax.ShapeDtypeStruct((n, d), jnp.float32),
        jax.ShapeDtypeStruct((n, _NUM_EMB), jnp.float32),
        jax.ShapeDtypeStruct((_K, 1, _R), jnp.int32),
        jax.ShapeDtypeStruct((1, 1), jnp.float32),
    )
    q, enc, idx, loss = pl.pallas_call(
        _vq_kernel,
        grid=(1,),
        in_specs=[
            pl.BlockSpec((n, d), lambda i: (0, 0)),
            pl.BlockSpec((_NUM_EMB, d), lambda i: (0, 0)),
        ],
        out_specs=(
            pl.BlockSpec(memory_space=pltpu.HBM),
            pl.BlockSpec(memory_space=pltpu.HBM),
            pl.BlockSpec((_K, 1, _R), lambda i: (0, 0, 0)),
            pl.BlockSpec((1, 1), lambda i: (0, 0)),
        ),
        out_shape=out_types,
        scratch_shapes=[
            pltpu.VMEM((_K, _R, _NUM_EMB), jnp.float32),
            pltpu.VMEM((_K, _R, _DIM), jnp.float32),
            pltpu.SemaphoreType.DMA((2, _K)),
        ],
    )(flat, W)
    return (q.reshape(b, l, d), loss.reshape(()),
            enc.reshape(b, l, _NUM_EMB), idx.reshape(b, l))


# speedup vs baseline: 1.2440x; 1.0291x over previous
import jax
import jax.numpy as jnp
from jax.experimental import pallas as pl
from jax.experimental.pallas import tpu as pltpu

_NUM_EMB = 1024
_DIM = 64
_N = 8 * 576
_K = 8
_R = _N // _K


def _vq_kernel(x_ref, w_ref, q_hbm, enc_hbm, idx_ref, loss_ref,
               enc_buf, q_buf, sem):
    w = w_ref[...]
    w2 = jnp.sum(w * w, axis=1)[None, :]

    counts = None
    mse = None
    for c in range(_K):
        xb = x_ref[c * _R:(c + 1) * _R, :]
        x2 = jnp.sum(xb * xb, axis=1, keepdims=True)
        xw = jax.lax.dot_general(xb, w, (((1,), (1,)), ((), ())),
                                 preferred_element_type=jnp.float32)
        d2 = x2 + w2 - 2.0 * xw

        idx = jnp.argmin(d2, axis=1).astype(jnp.int32)
        idx_ref[c, 0] = idx

        cols = jax.lax.broadcasted_iota(jnp.int32, d2.shape, 1)
        one_hot = (cols == idx[:, None]).astype(jnp.float32)
        enc_buf[c] = one_hot
        pltpu.make_async_copy(
            enc_buf.at[c], enc_hbm.at[pl.ds(c * _R, _R), :],
            sem.at[0, c]).start()

        q = jax.lax.dot_general(one_hot, w, (((1,), (0,)), ((), ())),
                                preferred_element_type=jnp.float32)
        q_buf[c] = q
        pltpu.make_async_copy(
            q_buf.at[c], q_hbm.at[pl.ds(c * _R, _R), :],
            sem.at[1, c]).start()

        diff = q - xb
        mse_part = jnp.sum(diff * diff).reshape(1, 1)
        counts_part = jnp.sum(one_hot, axis=0, keepdims=True)
        counts = counts_part if c == 0 else counts + counts_part
        mse = mse_part if c == 0 else mse + mse_part

    usage = counts * (1.0 / _N)
    du = usage - (1.0 / _NUM_EMB)
    usage_loss = jnp.sum(du * du) * (1.0 / _NUM_EMB)
    loss_ref[...] = 1.25 * (mse * (1.0 / (_N * _DIM))) + 0.1 * usage_loss

    for c in range(_K):
        pltpu.make_async_copy(
            enc_buf.at[c], enc_hbm.at[pl.ds(c * _R, _R), :],
            sem.at[0, c]).wait()
        pltpu.make_async_copy(
            q_buf.at[c], q_hbm.at[pl.ds(c * _R, _R), :],
            sem.at[1, c]).wait()


@jax.jit
def kernel(x, W):
    b, l, d = x.shape
    n = b * l
    flat = x.reshape(n, d)
    out_types = (
        jax.ShapeDtypeStruct((n, d), jnp.float32),
        jax.ShapeDtypeStruct((n, _NUM_EMB), jnp.float32),
        jax.ShapeDtypeStruct((_K, 1, _R), jnp.int32),
        jax.ShapeDtypeStruct((1, 1), jnp.float32),
    )
    q, enc, idx, loss = pl.pallas_call(
        _vq_kernel,
        grid=(1,),
        in_specs=[
            pl.BlockSpec((n, d), lambda i: (0, 0)),
            pl.BlockSpec((_NUM_EMB, d), lambda i: (0, 0)),
        ],
        out_specs=(
            pl.BlockSpec(memory_space=pltpu.HBM),
            pl.BlockSpec(memory_space=pltpu.HBM),
            pl.BlockSpec((_K, 1, _R), lambda i: (0, 0, 0)),
            pl.BlockSpec((1, 1), lambda i: (0, 0)),
        ),
        out_shape=out_types,
        scratch_shapes=[
            pltpu.VMEM((_K, _R, _NUM_EMB), jnp.float32),
            pltpu.VMEM((_K, _R, _DIM), jnp.float32),
            pltpu.SemaphoreType.DMA((2, _K)),
        ],
    )(flat, W)
    return (q.reshape(b, l, d), loss.reshape(()),
            enc.reshape(b, l, _NUM_EMB), idx.reshape(b, l))
